# baseline (speedup 1.0000x reference)
import jax
import jax.numpy as jnp
from jax.experimental import pallas as pl


def _body(emb_ref, out_ref):
    out_ref[...] = jnp.broadcast_to(emb_ref[...][None], out_ref.shape)


def kernel(input_seqs, pos_emb):
    B, L = input_seqs.shape
    Lk, E = pos_emb.shape
    BB = 64
    out = pl.pallas_call(
        _body,
        grid=(B // BB,),
        in_specs=[pl.BlockSpec((Lk, E), lambda i: (0, 0))],
        out_specs=pl.BlockSpec((BB, Lk, E), lambda i: (i, 0, 0)),
        out_shape=jax.ShapeDtypeStruct((B, Lk, E), jnp.float32),
    )(pos_emb)
    return out


# speedup vs baseline: 11.5143x; 11.5143x over previous
import jax
import jax.numpy as jnp
from jax.experimental import pallas as pl


def _body(emb_ref, out_ref):
    out_ref[...] = jnp.broadcast_to(emb_ref[...], out_ref.shape)


def kernel(input_seqs, pos_emb):
    B, L = input_seqs.shape
    Lk, E = pos_emb.shape
    BB = 512
    emb_t = pos_emb.reshape(Lk, E, 1)
    out_t = pl.pallas_call(
        _body,
        grid=(B // BB,),
        in_specs=[pl.BlockSpec((Lk, E, 1), lambda i: (0, 0, 0))],
        out_specs=pl.BlockSpec((Lk, E, BB), lambda i: (0, 0, i)),
        out_shape=jax.ShapeDtypeStruct((Lk, E, B), jnp.float32),
    )(emb_t)
    return jnp.transpose(out_t, (2, 0, 1))


# speedup vs baseline: 11.6426x; 1.0111x over previous
import jax
import jax.numpy as jnp
from jax.experimental import pallas as pl


def _body(emb_ref, out_ref):
    out_ref[...] = jnp.broadcast_to(emb_ref[...], out_ref.shape)


def kernel(input_seqs, pos_emb):
    B, L = input_seqs.shape
    Lk, E = pos_emb.shape
    BB = 256
    emb_t = pos_emb.reshape(Lk, E, 1)
    out_t = pl.pallas_call(
        _body,
        grid=(B // BB,),
        in_specs=[pl.BlockSpec((Lk, E, 1), lambda i: (0, 0, 0))],
        out_specs=pl.BlockSpec((Lk, E, BB), lambda i: (0, 0, i)),
        out_shape=jax.ShapeDtypeStruct((Lk, E, B), jnp.float32),
    )(emb_t)
    return jnp.transpose(out_t, (2, 0, 1))
